# baseline (speedup 1.0000x reference)
import numpy as np
import jax
import jax.numpy as jnp
from jax.experimental import pallas as pl
from jax.experimental.pallas import tpu as pltpu


def _outer_body(hc_ref, vr_ref, lhc_ref, lvr_ref, places_ref, sampled_ref):
    hc = hc_ref[0, 0]
    vr = vr_ref[0, 0]
    lhc = lhc_ref[0, 0]
    lvr = lvr_ref[0, 0]
    places_ref[0, 0] = hc * vr
    mh = (lhc == jnp.max(lhc, axis=0, keepdims=True)).astype(jnp.float32)
    mv = (lvr == jnp.max(lvr, axis=1, keepdims=True)).astype(jnp.float32)
    sampled_ref[0, 0] = ((hc * mh) * (vr * mv)) * 100.0


def kernel(x_cat):
    B, A, K = x_cat.shape
    P = A // 2
    f32 = jnp.float32

    t = 0
    beta = 0.1 + 0.1 * np.sin(t / 1000)
    nkey = jax.random.key(42)
    noise = jnp.stack(
        [beta * jax.random.gumbel(jax.random.fold_in(nkey, j), (B, K), dtype=f32)
         for j in range(A)], axis=1)

    h = x_cat[:, 0::2, :]
    v = x_cat[:, 1::2, :]
    log_h = jnp.log(h) + noise[:, 0::2, :]
    log_v = jnp.log(v) + noise[:, 1::2, :]

    hc = h[..., None]
    lhc = log_h[..., None]
    vr = v[:, :, None, :]
    lvr = log_v[:, :, None, :]

    col = pl.BlockSpec((1, 1, K, 1), lambda b, i: (b, i, 0, 0))
    row = pl.BlockSpec((1, 1, 1, K), lambda b, i: (b, i, 0, 0))
    out = pl.BlockSpec((1, 1, K, K), lambda b, i: (b, i, 0, 0))

    places, sampled = pl.pallas_call(
        _outer_body,
        grid=(B, P),
        in_specs=[col, row, col, row],
        out_specs=[out, out],
        out_shape=[jax.ShapeDtypeStruct((B, P, K, K), f32),
                   jax.ShapeDtypeStruct((B, P, K, K), f32)],
        compiler_params=pltpu.CompilerParams(
            dimension_semantics=("parallel", "parallel")),
    )(hc, vr, lhc, lvr)
    return (places, sampled)


# speedup vs baseline: 1.6012x; 1.6012x over previous
import numpy as np
import jax
import jax.numpy as jnp
from jax.experimental import pallas as pl
from jax.experimental.pallas import tpu as pltpu


def _outer_body(hc_ref, vr_ref, lhc_ref, lvr_ref, places_ref, sampled_ref):
    hc = hc_ref[0]
    vr = vr_ref[0]
    lhc = lhc_ref[0]
    lvr = lvr_ref[0]
    places_ref[0] = hc * vr
    mh = (lhc == jnp.max(lhc, axis=1, keepdims=True)).astype(jnp.float32)
    mv = (lvr == jnp.max(lvr, axis=2, keepdims=True)).astype(jnp.float32)
    sampled_ref[0] = ((hc * mh) * (vr * mv)) * 100.0


def kernel(x_cat):
    B, A, K = x_cat.shape
    P = A // 2
    f32 = jnp.float32

    t = 0
    beta = 0.1 + 0.1 * np.sin(t / 1000)
    nkey = jax.random.key(42)
    noise = jnp.stack(
        [beta * jax.random.gumbel(jax.random.fold_in(nkey, j), (B, K), dtype=f32)
         for j in range(A)], axis=1)

    h = x_cat[:, 0::2, :]
    v = x_cat[:, 1::2, :]
    log_h = jnp.log(h) + noise[:, 0::2, :]
    log_v = jnp.log(v) + noise[:, 1::2, :]

    hc = h[..., None]
    lhc = log_h[..., None]
    vr = v[:, :, None, :]
    lvr = log_v[:, :, None, :]

    col = pl.BlockSpec((1, P, K, 1), lambda b: (b, 0, 0, 0))
    row = pl.BlockSpec((1, P, 1, K), lambda b: (b, 0, 0, 0))
    out = pl.BlockSpec((1, P, K, K), lambda b: (b, 0, 0, 0))

    places, sampled = pl.pallas_call(
        _outer_body,
        grid=(B,),
        in_specs=[col, row, col, row],
        out_specs=[out, out],
        out_shape=[jax.ShapeDtypeStruct((B, P, K, K), f32),
                   jax.ShapeDtypeStruct((B, P, K, K), f32)],
        compiler_params=pltpu.CompilerParams(
            dimension_semantics=("parallel",)),
    )(hc, vr, lhc, lvr)
    return (places, sampled)
